# baseline (speedup 1.0000x reference)
import functools

import jax
import jax.numpy as jnp
from jax.experimental import pallas as pl
from jax.experimental.pallas import tpu as pltpu


def _fused_kernel(x_ref, w1_ref, b1_ref, w2_ref, b2_ref, v_ref, r_ref,
                  c_ref, out_ref, *, d2, nnz):
    iota = jax.lax.broadcasted_iota(jnp.int32, (d2, nnz), 0)
    rt = (r_ref[0:1, :] == iota).astype(jnp.float32)
    ct = (c_ref[0:1, :] == iota).astype(jnp.float32)
    st = jax.lax.dot_general(
        ct * v_ref[0:1, :], rt,
        (((1,), (1,)), ((), ())),
        preferred_element_type=jnp.float32)
    w12 = jnp.dot(w1_ref[...], w2_ref[...],
                  preferred_element_type=jnp.float32)
    wf = jnp.dot(w12, st, preferred_element_type=jnp.float32)
    bf = jnp.dot(
        jnp.dot(b1_ref[...], w2_ref[...],
                preferred_element_type=jnp.float32) + b2_ref[...],
        st, preferred_element_type=jnp.float32)

    out_ref[...] = jnp.dot(x_ref[...], wf,
                           preferred_element_type=jnp.float32) + bf


@jax.jit
def kernel(inputs, W1, b1, W2, b2, sp_vals, sp_rows, sp_cols):
    batch, d_in = inputs.shape
    d1 = W1.shape[1]
    d2 = W2.shape[1]
    nnz = sp_vals.shape[0]

    blk = 16384
    grid = (batch // blk,)

    full = lambda shape: pl.BlockSpec(shape, lambda i: (0, 0))
    out = pl.pallas_call(
        functools.partial(_fused_kernel, d2=d2, nnz=nnz),
        grid=grid,
        in_specs=[
            pl.BlockSpec((blk, d_in), lambda i: (i, 0)),
            full((d_in, d1)),
            full((1, d1)),
            full((d1, d2)),
            full((1, d2)),
            full((1, nnz)),
            full((1, nnz)),
            full((1, nnz)),
        ],
        out_specs=pl.BlockSpec((blk, d2), lambda i: (i, 0)),
        out_shape=jax.ShapeDtypeStruct((batch, d2), jnp.float32),
        compiler_params=pltpu.CompilerParams(
            dimension_semantics=("parallel",)),
    )(inputs, W1, b1.reshape(1, d1), W2, b2.reshape(1, d2),
      sp_vals.reshape(1, nnz), sp_rows.reshape(1, nnz),
      sp_cols.reshape(1, nnz))
    return out


# speedup vs baseline: 1.4855x; 1.4855x over previous
import functools

import jax
import jax.numpy as jnp
from jax.experimental import pallas as pl
from jax.experimental.pallas import tpu as pltpu


def _fused_kernel(x_ref, w1_ref, b1_ref, w2_ref, b2_ref, v_ref, r_ref,
                  c_ref, out_ref, *, d2, nnz):
    iota = jax.lax.broadcasted_iota(jnp.int32, (d2, nnz), 0)
    rt = (r_ref[0:1, :] == iota).astype(jnp.float32)
    ct = (c_ref[0:1, :] == iota).astype(jnp.float32)
    st = jax.lax.dot_general(
        ct * v_ref[0:1, :], rt,
        (((1,), (1,)), ((), ())),
        preferred_element_type=jnp.float32)
    w12 = jnp.dot(w1_ref[...], w2_ref[...],
                  preferred_element_type=jnp.float32)
    wf = jnp.dot(w12, st, preferred_element_type=jnp.float32)
    bvec = jnp.dot(b1_ref[...], w2_ref[...],
                   preferred_element_type=jnp.float32) + b2_ref[...]
    bf_col = jax.lax.dot_general(
        st, bvec,
        (((0,), (1,)), ((), ())),
        preferred_element_type=jnp.float32)

    y_t = jax.lax.dot_general(
        wf, x_ref[...],
        (((0,), (1,)), ((), ())),
        preferred_element_type=jnp.float32)
    out_ref[...] = y_t + bf_col


@jax.jit
def kernel(inputs, W1, b1, W2, b2, sp_vals, sp_rows, sp_cols):
    batch, d_in = inputs.shape
    d1 = W1.shape[1]
    d2 = W2.shape[1]
    nnz = sp_vals.shape[0]

    blk = 2048
    grid = (batch // blk,)

    full = lambda shape: pl.BlockSpec(shape, lambda i: (0, 0))
    out = pl.pallas_call(
        functools.partial(_fused_kernel, d2=d2, nnz=nnz),
        grid=grid,
        in_specs=[
            pl.BlockSpec((blk, d_in), lambda i: (i, 0)),
            full((d_in, d1)),
            full((1, d1)),
            full((d1, d2)),
            full((1, d2)),
            full((1, nnz)),
            full((1, nnz)),
            full((1, nnz)),
        ],
        out_specs=pl.BlockSpec((d2, blk), lambda i: (0, i)),
        out_shape=jax.ShapeDtypeStruct((d2, batch), jnp.float32),
        compiler_params=pltpu.CompilerParams(
            dimension_semantics=("parallel",)),
    )(inputs, W1, b1.reshape(1, d1), W2, b2.reshape(1, d2),
      sp_vals.reshape(1, nnz), sp_rows.reshape(1, nnz),
      sp_cols.reshape(1, nnz))
    return out.T


# speedup vs baseline: 1.9631x; 1.3215x over previous
import functools

import jax
import jax.numpy as jnp
from jax.experimental import pallas as pl
from jax.experimental.pallas import tpu as pltpu


def _fused_kernel(x_ref, w1_ref, b1_ref, w2_ref, b2_ref, v_ref, r_ref,
                  c_ref, out_ref, *, d2, nnz):
    iota = jax.lax.broadcasted_iota(jnp.int32, (d2, nnz), 0)
    rt = (r_ref[0:1, :] == iota).astype(jnp.float32)
    ct = (c_ref[0:1, :] == iota).astype(jnp.float32)
    st = jax.lax.dot_general(
        ct * v_ref[0:1, :], rt,
        (((1,), (1,)), ((), ())),
        preferred_element_type=jnp.float32)
    w12 = jnp.dot(w1_ref[...], w2_ref[...],
                  preferred_element_type=jnp.float32)
    wf = jnp.dot(w12, st, preferred_element_type=jnp.float32)
    bvec = jnp.dot(b1_ref[...], w2_ref[...],
                   preferred_element_type=jnp.float32) + b2_ref[...]
    bf_col = jax.lax.dot_general(
        st, bvec,
        (((0,), (1,)), ((), ())),
        preferred_element_type=jnp.float32)

    y_t = jax.lax.dot_general(
        wf, x_ref[...],
        (((0,), (1,)), ((), ())),
        preferred_element_type=jnp.float32)
    out_ref[...] = y_t + bf_col


@jax.jit
def kernel(inputs, W1, b1, W2, b2, sp_vals, sp_rows, sp_cols):
    batch, d_in = inputs.shape
    d1 = W1.shape[1]
    d2 = W2.shape[1]
    nnz = sp_vals.shape[0]

    blk = 4096
    grid = (batch // blk,)

    full = lambda shape: pl.BlockSpec(shape, lambda i: (0, 0))
    out = pl.pallas_call(
        functools.partial(_fused_kernel, d2=d2, nnz=nnz),
        grid=grid,
        in_specs=[
            pl.BlockSpec((blk, d_in), lambda i: (i, 0)),
            full((d_in, d1)),
            full((1, d1)),
            full((d1, d2)),
            full((1, d2)),
            full((1, nnz)),
            full((1, nnz)),
            full((1, nnz)),
        ],
        out_specs=pl.BlockSpec((d2, blk), lambda i: (0, i)),
        out_shape=jax.ShapeDtypeStruct((d2, batch), jnp.float32),
        compiler_params=pltpu.CompilerParams(
            dimension_semantics=("parallel",)),
    )(inputs, W1, b1.reshape(1, d1), W2, b2.reshape(1, d2),
      sp_vals.reshape(1, nnz), sp_rows.reshape(1, nnz),
      sp_cols.reshape(1, nnz))
    return out.T


# speedup vs baseline: 2.2790x; 1.1609x over previous
import functools

import jax
import jax.numpy as jnp
from jax.experimental import pallas as pl
from jax.experimental.pallas import tpu as pltpu


def _fused_kernel(x_ref, w1_ref, b1_ref, w2_ref, b2_ref, v_ref, r_ref,
                  c_ref, out_ref, *, d2, nnz):
    iota = jax.lax.broadcasted_iota(jnp.int32, (d2, nnz), 0)
    rt = (r_ref[0:1, :] == iota).astype(jnp.float32)
    ct = (c_ref[0:1, :] == iota).astype(jnp.float32)
    st = jax.lax.dot_general(
        ct * v_ref[0:1, :], rt,
        (((1,), (1,)), ((), ())),
        preferred_element_type=jnp.float32)
    w12 = jnp.dot(w1_ref[...], w2_ref[...],
                  preferred_element_type=jnp.float32)
    wf = jnp.dot(w12, st, preferred_element_type=jnp.float32)
    bvec = jnp.dot(b1_ref[...], w2_ref[...],
                   preferred_element_type=jnp.float32) + b2_ref[...]
    bf_col = jax.lax.dot_general(
        st, bvec,
        (((0,), (1,)), ((), ())),
        preferred_element_type=jnp.float32)

    y_t = jax.lax.dot_general(
        wf, x_ref[...],
        (((0,), (1,)), ((), ())),
        preferred_element_type=jnp.float32)
    out_ref[...] = y_t + bf_col


@jax.jit
def kernel(inputs, W1, b1, W2, b2, sp_vals, sp_rows, sp_cols):
    batch, d_in = inputs.shape
    d1 = W1.shape[1]
    d2 = W2.shape[1]
    nnz = sp_vals.shape[0]

    blk = 8192
    grid = (batch // blk,)

    full = lambda shape: pl.BlockSpec(shape, lambda i: (0, 0))
    out = pl.pallas_call(
        functools.partial(_fused_kernel, d2=d2, nnz=nnz),
        grid=grid,
        in_specs=[
            pl.BlockSpec((blk, d_in), lambda i: (i, 0)),
            full((d_in, d1)),
            full((1, d1)),
            full((d1, d2)),
            full((1, d2)),
            full((1, nnz)),
            full((1, nnz)),
            full((1, nnz)),
        ],
        out_specs=pl.BlockSpec((d2, blk), lambda i: (0, i)),
        out_shape=jax.ShapeDtypeStruct((d2, batch), jnp.float32),
        compiler_params=pltpu.CompilerParams(
            dimension_semantics=("parallel",)),
    )(inputs, W1, b1.reshape(1, d1), W2, b2.reshape(1, d2),
      sp_vals.reshape(1, nnz), sp_rows.reshape(1, nnz),
      sp_cols.reshape(1, nnz))
    return out.T
